# baseline (speedup 1.0000x reference)
import functools

import numpy as np
import jax
import jax.numpy as jnp
from jax import lax
from jax.experimental import pallas as pl
from jax.experimental.pallas import tpu as pltpu
from jax.experimental.pallas import tpu_sc as plsc

B, S, V, D, E = 32, 512, 512, 128, 8
CH = 32
NCHUNK = S // CH
LANES = 16

_PERM = np.empty((V,), dtype=np.int32)
for _g in range(V // 32):
    for _i in range(16):
        _PERM[_g * 32 + 2 * _i] = _g * 32 + _i
        _PERM[_g * 32 + 2 * _i + 1] = _g * 32 + 16 + _i


def _tc_body(x_ref, rw_ref, rb_ref, emb0_ref, woutr_ref, bout_ref,
             emb_ref, wout_ref,
             idx0_ref, idx1_ref, w0_ref, w1_ref, q_ref, t_sc):
    @pl.when(pl.program_id(0) == 0)
    def _router():
        xi = x_ref[...]
        xf = xi.astype(jnp.float32)
        logits = (jnp.dot(xf, rw_ref[...], preferred_element_type=jnp.float32)
                  + rb_ref[...])
        iota = lax.broadcasted_iota(jnp.int32, (B, E), 1)
        m0 = jnp.max(logits, axis=1, keepdims=True)
        a0 = jnp.min(jnp.where(logits == m0, iota, E), axis=1)
        masked = jnp.where(iota == a0[:, None], -1e30, logits)
        m1 = jnp.max(masked, axis=1, keepdims=True)
        a1 = jnp.min(jnp.where(masked == m1, iota, E), axis=1)
        d = jnp.exp(m1[:, 0] - m0[:, 0])
        w0 = 1.0 / (1.0 + d)
        idx0_ref[...] = a0[:, None] * V + xi
        idx1_ref[...] = a1[:, None] * V + xi
        w0_ref[...] = jnp.broadcast_to(w0[:, None], (B, 128))
        w1_ref[...] = jnp.broadcast_to((1.0 - w0)[:, None], (B, 128))
        t_sc[...] = (jnp.dot(emb0_ref[...], woutr_ref[...],
                             preferred_element_type=jnp.float32)
                     + jnp.sum(bout_ref[...], axis=0, keepdims=True))

    eb = emb_ref[0]
    h = eb - eb[0:1, :]
    q_ref[0] = (jnp.dot(h, wout_ref[0], preferred_element_type=jnp.float32)
                + t_sc[...]).astype(jnp.bfloat16)


def _tc_call(x, router_W, router_b, emb0, woutr, bout, emb, Wout):
    return pl.pallas_call(
        _tc_body,
        grid=(E,),
        in_specs=[
            pl.BlockSpec((B, S), lambda e: (0, 0)),
            pl.BlockSpec((S, E), lambda e: (0, 0)),
            pl.BlockSpec((1, E), lambda e: (0, 0)),
            pl.BlockSpec((1, E * D), lambda e: (0, 0)),
            pl.BlockSpec((E * D, V), lambda e: (0, 0)),
            pl.BlockSpec((E, V), lambda e: (0, 0)),
            pl.BlockSpec((1, V, D), lambda e: (e, 0, 0)),
            pl.BlockSpec((1, D, V), lambda e: (e, 0, 0)),
        ],
        out_specs=(
            pl.BlockSpec((B, S), lambda e: (0, 0)),
            pl.BlockSpec((B, S), lambda e: (0, 0)),
            pl.BlockSpec((B, 128), lambda e: (0, 0)),
            pl.BlockSpec((B, 128), lambda e: (0, 0)),
            pl.BlockSpec((1, V, V), lambda e: (e, 0, 0)),
        ),
        out_shape=(
            jax.ShapeDtypeStruct((B, S), jnp.int32),
            jax.ShapeDtypeStruct((B, S), jnp.int32),
            jax.ShapeDtypeStruct((B, 128), jnp.float32),
            jax.ShapeDtypeStruct((B, 128), jnp.float32),
            jax.ShapeDtypeStruct((E, V, V), jnp.bfloat16),
        ),
        scratch_shapes=[pltpu.VMEM((1, V), jnp.float32)],
    )(x, router_W, router_b, emb0, woutr, bout, emb, Wout)


def _sc_body(q_hbm, idx0_hbm, idx1_hbm, w0_hbm, w1_hbm, out_hbm,
             idx0_v, idx1_v, w0_v, w1_v, buf0, buf1, outb,
             g0a, g0b, g1a, g1b, wa, wb):
    nc = 2
    b = lax.axis_index("s") * nc + lax.axis_index("c")
    pltpu.sync_copy(idx0_hbm.at[b], idx0_v)
    pltpu.sync_copy(idx1_hbm.at[b], idx1_v)
    pltpu.sync_copy(w0_hbm.at[b, pl.ds(0, LANES)], w0_v)
    pltpu.sync_copy(w1_hbm.at[b, pl.ds(0, LANES)], w1_v)
    w0 = w0_v[...]
    w1 = w1_v[...]
    gsem = [(g0a, g1a), (g0b, g1b)]
    wsem = [wa, wb]

    def issue(c):
        s = c % 2
        cp0 = pltpu.async_copy(
            q_hbm.at[idx0_v.at[pl.ds(c * CH, CH)]], buf0.at[s], gsem[s][0])
        cp1 = pltpu.async_copy(
            q_hbm.at[idx1_v.at[pl.ds(c * CH, CH)]], buf1.at[s], gsem[s][1])
        return cp0, cp1

    pending = {0: issue(0)}
    writes = {}
    for c in range(NCHUNK):
        s = c % 2
        if c + 1 < NCHUNK:
            pending[c + 1] = issue(c + 1)
        cp0, cp1 = pending.pop(c)
        cp0.wait()
        cp1.wait()
        if c >= 2:
            writes.pop(c - 2).wait()

        def row_body(t, carry, s=s):
            for j in range(V // 32):
                sl = pl.ds(j * LANES, LANES)
                v0 = buf0[s, t, sl]
                v1 = buf1[s, t, sl]
                lo0 = lax.bitcast_convert_type(v0 << 16, jnp.float32)
                hi0 = lax.bitcast_convert_type(v0 & -65536, jnp.float32)
                lo1 = lax.bitcast_convert_type(v1 << 16, jnp.float32)
                hi1 = lax.bitcast_convert_type(v1 & -65536, jnp.float32)
                outb[s, t, pl.ds(j * 32, LANES)] = lo0 * w0 + lo1 * w1
                outb[s, t, pl.ds(j * 32 + LANES, LANES)] = hi0 * w0 + hi1 * w1
            return carry

        lax.fori_loop(0, CH, row_body, 0, unroll=False)
        writes[c] = pltpu.async_copy(
            outb.at[s], out_hbm.at[pl.ds(b * S + c * CH, CH)], wsem[s])
    writes.pop(NCHUNK - 2).wait()
    writes.pop(NCHUNK - 1).wait()


def _sc_call(qf, idx0, idx1, w0r, w1r):
    mesh = plsc.VectorSubcoreMesh(core_axis_name="c", subcore_axis_name="s")
    f = functools.partial(
        pl.kernel,
        mesh=mesh,
        out_type=jax.ShapeDtypeStruct((B * S, V), jnp.float32),
        scratch_types=[
            pltpu.VMEM((S,), jnp.int32),
            pltpu.VMEM((S,), jnp.int32),
            pltpu.VMEM((LANES,), jnp.float32),
            pltpu.VMEM((LANES,), jnp.float32),
            pltpu.VMEM((2, CH, V // 2), jnp.int32),
            pltpu.VMEM((2, CH, V // 2), jnp.int32),
            pltpu.VMEM((2, CH, V), jnp.float32),
            pltpu.SemaphoreType.DMA,
            pltpu.SemaphoreType.DMA,
            pltpu.SemaphoreType.DMA,
            pltpu.SemaphoreType.DMA,
            pltpu.SemaphoreType.DMA,
            pltpu.SemaphoreType.DMA,
        ],
    )(_sc_body)
    return f(qf, idx0, idx1, w0r, w1r)


def kernel(x, router_W, router_b, emb, Wout, bout):
    perm = jnp.asarray(_PERM)
    Woutp = Wout[:, :, perm]
    boutp = bout[:, perm]
    emb0 = emb[:, 0, :].reshape(1, E * D)
    woutr = Woutp.reshape(E * D, V)
    idx0, idx1, w0r, w1r, q = _tc_call(
        x, router_W, router_b.reshape(1, E), emb0, woutr, boutp, emb, Woutp)
    qi = lax.bitcast_convert_type(
        q.reshape(E * V, V // 2, 2), jnp.int32)
    out = _sc_call(qi, idx0, idx1, w0r, w1r)
    return out.reshape(B, S, V)
